# baseline (speedup 1.0000x reference)
import dataclasses
import functools

import jax
import jax.numpy as jnp
from jax import lax
from jax.experimental import pallas as pl
from jax.experimental.pallas import tpu as pltpu
from jax.experimental.pallas import tpu_sc as plsc

NUM_CORES = 2
NUM_SUBCORES = 16
NUM_WORKERS = NUM_CORES * NUM_SUBCORES
LANES = 16
CHUNK = 128


def kernel(item_ids, context_ids, user_emb, item_emb, user_bias, item_bias):
    batch = item_ids.shape[0]
    dim = user_emb.shape[1]
    bpw = batch // NUM_WORKERS
    n_chunks = bpw // CHUNK

    ii = item_ids.astype(jnp.int32)
    ci = context_ids.astype(jnp.int32)

    mesh = plsc.VectorSubcoreMesh(core_axis_name="c", subcore_axis_name="s")
    cp = pltpu.CompilerParams()
    if "needs_layout_passes" in pltpu.CompilerParams.__dataclass_fields__:
        cp = dataclasses.replace(cp, needs_layout_passes=False)

    @functools.partial(
        pl.kernel,
        out_type=jax.ShapeDtypeStruct((batch,), jnp.float32),
        mesh=mesh,
        compiler_params=cp,
        scratch_types=[
            pltpu.VMEM((bpw,), jnp.int32),
            pltpu.VMEM((bpw,), jnp.int32),
            pltpu.VMEM((CHUNK, dim), jnp.float32),
            pltpu.VMEM((CHUNK, dim), jnp.float32),
            pltpu.VMEM((CHUNK, 1), jnp.float32),
            pltpu.VMEM((CHUNK, 1), jnp.float32),
            pltpu.VMEM((bpw,), jnp.float32),
            pltpu.VMEM((LANES * LANES,), jnp.float32),
            pltpu.SemaphoreType.DMA,
            pltpu.SemaphoreType.DMA,
        ],
    )
    def glove_kernel(ii_hbm, ci_hbm, ue_hbm, ie_hbm, ub_hbm, ib_hbm, out_hbm,
                     idx_u, idx_i, rows_u, rows_i, sb_u, sb_i,
                     out_v, tr_buf, isem, sem):
        wid = lax.axis_index("s") * NUM_CORES + lax.axis_index("c")
        base = wid * bpw

        pltpu.async_copy(ii_hbm.at[pl.ds(base, bpw)], idx_u, isem).wait()
        pltpu.async_copy(ci_hbm.at[pl.ds(base, bpw)], idx_i, isem).wait()

        lane_iota = lax.iota(jnp.int32, LANES)
        zero_iota = lane_iota * 0

        @pl.loop(0, n_chunks)
        def _(k):
            ch = k * CHUNK

            @pl.loop(0, CHUNK, step=LANES)
            def _(q):
                ivec_u = idx_u[pl.ds(ch + q, LANES)]
                ivec_i = idx_i[pl.ds(ch + q, LANES)]
                for r16 in range(LANES):
                    iu = ivec_u[r16]
                    iv = ivec_i[r16]
                    r = q + r16
                    pltpu.async_copy(ue_hbm.at[iu], rows_u.at[r], sem)
                    pltpu.async_copy(ie_hbm.at[iv], rows_i.at[r], sem)
                    pltpu.async_copy(ub_hbm.at[iu], sb_u.at[r], sem)
                    pltpu.async_copy(ib_hbm.at[iv], sb_i.at[r], sem)

            @pl.loop(0, CHUNK)
            def _(r):
                pltpu.make_async_copy(ue_hbm.at[0], rows_u.at[r], sem).wait()
                pltpu.make_async_copy(ie_hbm.at[0], rows_i.at[r], sem).wait()
                pltpu.make_async_copy(ub_hbm.at[0], sb_u.at[r], sem).wait()
                pltpu.make_async_copy(ib_hbm.at[0], sb_i.at[r], sem).wait()

            @pl.loop(0, CHUNK, step=LANES)
            def _(blk):
                for r16 in range(LANES):
                    acc = (rows_u[blk + r16, pl.ds(0, LANES)]
                           * rows_i[blk + r16, pl.ds(0, LANES)])
                    for c in range(LANES, dim, LANES):
                        acc = acc + (rows_u[blk + r16, pl.ds(c, LANES)]
                                     * rows_i[blk + r16, pl.ds(c, LANES)])
                    plsc.store_scatter(tr_buf, [lane_iota * LANES + r16], acc)
                s = (plsc.load_gather(sb_u, [blk + lane_iota, zero_iota])
                     + plsc.load_gather(sb_i, [blk + lane_iota, zero_iota]))
                for l in range(LANES):
                    s = s + tr_buf[pl.ds(l * LANES, LANES)]
                out_v[pl.ds(ch + blk, LANES)] = s

        pltpu.sync_copy(out_v, out_hbm.at[pl.ds(base, bpw)])

    return glove_kernel(ii, ci, user_emb, item_emb, user_bias, item_bias)


# speedup vs baseline: 1.0297x; 1.0297x over previous
import dataclasses
import functools

import jax
import jax.numpy as jnp
from jax import lax
from jax.experimental import pallas as pl
from jax.experimental.pallas import tpu as pltpu
from jax.experimental.pallas import tpu_sc as plsc

NUM_CORES = 2
NUM_SUBCORES = 16
NUM_WORKERS = NUM_CORES * NUM_SUBCORES
LANES = 16
IDX_ROW = 128


def kernel(item_ids, context_ids, user_emb, item_emb, user_bias, item_bias):
    batch = item_ids.shape[0]
    dim = user_emb.shape[1]
    bpw = batch // NUM_WORKERS
    n_chunks = bpw // IDX_ROW

    ii = item_ids.astype(jnp.int32).reshape(NUM_WORKERS * n_chunks, IDX_ROW)
    ci = context_ids.astype(jnp.int32).reshape(NUM_WORKERS * n_chunks, IDX_ROW)
    one = (item_ids[0] * 0 + 1).astype(jnp.float32)
    ue = user_emb * one
    ub = user_bias.reshape(-1)
    ib = item_bias.reshape(-1)

    mesh = plsc.VectorSubcoreMesh(core_axis_name="c", subcore_axis_name="s")
    cp = pltpu.CompilerParams()
    if "needs_layout_passes" in pltpu.CompilerParams.__dataclass_fields__:
        cp = dataclasses.replace(cp, needs_layout_passes=False)
    if "use_tc_tiling_on_sc" in pltpu.CompilerParams.__dataclass_fields__:
        cp = dataclasses.replace(cp, use_tc_tiling_on_sc=False)

    @functools.partial(
        pl.kernel,
        out_type=jax.ShapeDtypeStruct((batch,), jnp.float32),
        mesh=mesh,
        compiler_params=cp,
        scratch_types=[
            pltpu.VMEM((n_chunks, IDX_ROW), jnp.int32),
            pltpu.VMEM((n_chunks, IDX_ROW), jnp.int32),
            pltpu.VMEM((bpw, dim), jnp.float32),
            pltpu.VMEM((bpw, dim), jnp.float32),
            pltpu.VMEM((bpw,), jnp.float32),
            pltpu.VMEM((bpw,), jnp.float32),
            pltpu.VMEM((bpw,), jnp.float32),
            pltpu.VMEM((LANES * LANES,), jnp.float32),
            pltpu.SemaphoreType.DMA,
        ],
    )
    def glove_kernel(ii_hbm, ci_hbm, ue_hbm, ie_hbm, ub_hbm, ib_hbm, out_hbm,
                     idx_u, idx_i, rows_u, rows_i, bias_u, bias_i, out_v,
                     tr_buf, sem):
        wid = lax.axis_index("s") * NUM_CORES + lax.axis_index("c")
        base = wid * bpw

        pltpu.sync_copy(ii_hbm.at[pl.ds(wid * n_chunks, n_chunks)], idx_u)
        pltpu.sync_copy(ci_hbm.at[pl.ds(wid * n_chunks, n_chunks)], idx_i)

        copies = []
        for j in range(n_chunks):
            copies.append(pltpu.async_copy(
                ue_hbm.at[idx_u.at[j]], rows_u.at[pl.ds(j * IDX_ROW, IDX_ROW)], sem))
            copies.append(pltpu.async_copy(
                ie_hbm.at[idx_i.at[j]], rows_i.at[pl.ds(j * IDX_ROW, IDX_ROW)], sem))
            copies.append(pltpu.async_copy(
                ub_hbm.at[idx_u.at[j]], bias_u.at[pl.ds(j * IDX_ROW, IDX_ROW)], sem))
            copies.append(pltpu.async_copy(
                ib_hbm.at[idx_i.at[j]], bias_i.at[pl.ds(j * IDX_ROW, IDX_ROW)], sem))
        for c in copies:
            c.wait()

        lane_iota = lax.iota(jnp.int32, LANES)

        @pl.loop(0, bpw, step=LANES)
        def _(blk):
            for r16 in range(LANES):
                acc = (rows_u[blk + r16, pl.ds(0, LANES)]
                       * rows_i[blk + r16, pl.ds(0, LANES)])
                for c in range(LANES, dim, LANES):
                    acc = acc + (rows_u[blk + r16, pl.ds(c, LANES)]
                                 * rows_i[blk + r16, pl.ds(c, LANES)])
                plsc.store_scatter(tr_buf, [lane_iota * LANES + r16], acc)
            s = bias_u[pl.ds(blk, LANES)] + bias_i[pl.ds(blk, LANES)]
            for l in range(LANES):
                s = s + tr_buf[pl.ds(l * LANES, LANES)]
            out_v[pl.ds(blk, LANES)] = s

        pltpu.sync_copy(out_v, out_hbm.at[pl.ds(base, bpw)])

    return glove_kernel(ii, ci, ue, item_emb, ub, ib)
